# Initial kernel scaffold; baseline (speedup 1.0000x reference)
import jax
import jax.numpy as jnp
from jax.experimental import pallas as pl


def kernel(x, adj, adj_attr, pass_adj, pass_adj_attr, train_adj, train_adj_attr, batch, W1, b1, W2, b2, W3, b3, L1W, L1b, L2W, L2b, L3W, L3b):
    raise NotImplementedError("write your pallas kernel here")


# speedup vs baseline: 2.8347x; 2.8347x over previous
import functools

import jax
import jax.numpy as jnp
from jax import lax
from jax.experimental import pallas as pl
from jax.experimental.pallas import tpu as pltpu
from jax.experimental.pallas import tpu_sc as plsc

N = 10000
E = 160000
F = 256
H = 256
G = 64

TILES = 16
EPT_PAD = 10240
CHUNKS = EPT_PAD // 128
ROWS_PT = N // TILES


def _mm_body(x_ref, w_ref, b_ref, o_ref):
    o_ref[...] = jnp.dot(x_ref[...] + b_ref[...], w_ref[...],
                         preferred_element_type=jnp.float32)


def _tc_mm(x, W, b):
    return pl.pallas_call(
        _mm_body,
        grid=(25,),
        in_specs=[
            pl.BlockSpec((400, 256), lambda i: (i, 0)),
            pl.BlockSpec((256, 256), lambda i: (0, 0)),
            pl.BlockSpec((1, 256), lambda i: (0, 0)),
        ],
        out_specs=pl.BlockSpec((400, 256), lambda i: (i, 0)),
        out_shape=jax.ShapeDtypeStruct((N, 256), jnp.float32),
    )(x, W, b)


def _head_body(y_ref, bt_ref, b3_ref, L1W, L1b, L2W, L2b, L3W, L3b,
               o_ref, g_acc):
    i = pl.program_id(0)
    oh = (bt_ref[...] == lax.broadcasted_iota(jnp.int32, (400, 64), 1)
          ).astype(jnp.float32)
    part = lax.dot_general(oh, y_ref[...] + b3_ref[...],
                           (((0,), (0,)), ((), ())),
                           preferred_element_type=jnp.float32,
                           precision=lax.Precision.HIGHEST)

    @pl.when(i == 0)
    def _():
        g_acc[...] = part

    @pl.when(i > 0)
    def _():
        g_acc[...] = g_acc[...] + part

    @pl.when(i == 24)
    def _():
        dot = functools.partial(jnp.dot, preferred_element_type=jnp.float32)
        g = dot(g_acc[...], L1W[...]) + L1b[...]
        g = dot(g, L2W[...]) + L2b[...]
        o_ref[...] = dot(g, L3W[...]) + L3b[...]


def _tc_head(y3, batch2, b3, L1W, L1b, L2W, L2b, L3W, L3b):
    const = lambda shape: pl.BlockSpec(shape, lambda i: (0, 0))
    return pl.pallas_call(
        _head_body,
        grid=(25,),
        in_specs=[
            pl.BlockSpec((400, 256), lambda i: (i, 0)),
            pl.BlockSpec((400, 1), lambda i: (i, 0)),
            const((1, 256)),
            const((256, 256)), const((1, 256)),
            const((256, 256)), const((1, 256)),
            const((256, 1)), const((1, 1)),
        ],
        out_specs=pl.BlockSpec((64, 1), lambda i: (0, 0)),
        out_shape=jax.ShapeDtypeStruct((64, 1), jnp.float32),
        scratch_shapes=[pltpu.VMEM((64, 256), jnp.float32)],
    )(y3, batch2, b3, L1W, L1b, L2W, L2b, L3W, L3b)


def _agg_body(h_hbm, src_hbm, dst_hbm, w_hbm, out_hbm,
              esrc, edst, ew, idxbuf, rows_v, sem, y_sp):
    cid = lax.axis_index("c")
    wid = lax.axis_index("s")

    pltpu.sync_copy(src_hbm.at[wid], esrc)
    pltpu.sync_copy(dst_hbm.at[wid], edst)
    pltpu.sync_copy(w_hbm.at[wid], ew)

    zero16 = jnp.zeros((16,), jnp.float32)

    @pl.loop(0, 128)
    def _z(r):
        for j in range(8):
            rows_v[r, pl.ds(j * 16, 16)] = zero16

    @pl.loop(0, 5)
    def _zero_strip(k):
        pltpu.sync_copy(rows_v.at[pl.ds(0, 125)],
                        y_sp.at[pl.ds(wid * ROWS_PT + k * 125, 125)])

    plsc.subcore_barrier()

    iota = lax.iota(jnp.int32, 16)

    @pl.loop(0, CHUNKS)
    def _chunk(i):
        for j in range(8):
            sv = esrc[i, pl.ds(j * 16, 16)]
            idxbuf[pl.ds(j * 16, 16)] = sv * 2 + cid
        pltpu.async_copy(h_hbm.at[idxbuf], rows_v, sem).wait()

        @pl.loop(0, 8)
        def _scale(j):
            wv = ew[i, pl.ds(j * 16, 16)]
            for e in range(16):
                ws = jnp.take(wv, iota * 0 + e)
                r = j * 16 + e
                for k in range(8):
                    rows_v[r, pl.ds(k * 16, 16)] = (
                        rows_v[r, pl.ds(k * 16, 16)] * ws)

        pltpu.async_copy(rows_v, y_sp.at[edst.at[i]], sem, add=True).wait()

    plsc.subcore_barrier()

    pltpu.sync_copy(y_sp.at[pl.ds(wid * ROWS_PT, ROWS_PT)],
                    out_hbm.at[pl.ds(wid * ROWS_PT, ROWS_PT), cid])


def _sc_agg(h2v, src3, dst3, w3):
    f32, i32 = jnp.float32, jnp.int32
    mesh = plsc.VectorSubcoreMesh(core_axis_name="c", subcore_axis_name="s",
                                  num_cores=2, num_subcores=TILES)
    kern = pl.kernel(
        _agg_body,
        out_type=jax.ShapeDtypeStruct((N, 2, 128), f32),
        mesh=mesh,
        compiler_params=pltpu.CompilerParams(
            use_tc_tiling_on_sc=False, needs_layout_passes=False),
        scratch_types=[
            pltpu.VMEM((CHUNKS, 128), i32),
            pltpu.VMEM((CHUNKS, 128), i32),
            pltpu.VMEM((CHUNKS, 128), f32),
            pltpu.VMEM((128,), i32),
            pltpu.VMEM((128, 128), f32),
            pltpu.SemaphoreType.DMA,
            pltpu.VMEM_SHARED((N, 128), f32),
        ],
    )
    return kern(h2v, src3, dst3, w3)


def kernel(x, adj, adj_attr, pass_adj, pass_adj_attr, train_adj,
           train_adj_attr, batch, W1, b1, W2, b2, W3, b3,
           L1W, L1b, L2W, L2b, L3W, L3b):
    f32, i32 = jnp.float32, jnp.int32

    def prep(ei, ew):
        def lay(a, dt):
            a = a.astype(dt).reshape(TILES, E // TILES)
            return jnp.pad(a, ((0, 0), (0, EPT_PAD - E // TILES))).reshape(
                TILES, CHUNKS, 128)
        return lay(ei[0], i32), lay(ei[1], i32), lay(ew, f32)

    edges = [prep(train_adj, train_adj_attr),
             prep(adj, adj_attr),
             prep(pass_adj, pass_adj_attr)]
    biases = [jnp.zeros((1, H), f32), b1.reshape(1, H), b2.reshape(1, H)]
    weights = [W1, W2, W3]

    h = x
    for l in range(3):
        Hk = _tc_mm(h, weights[l], biases[l])
        s3, d3, w3 = edges[l]
        y = _sc_agg(Hk.reshape(2 * N, 128), s3, d3, w3)
        h = y.reshape(N, 256)

    out = _tc_head(h, batch.astype(i32).reshape(N, 1), b3.reshape(1, H),
                   L1W, L1b.reshape(1, H), L2W, L2b.reshape(1, H),
                   L3W, L3b.reshape(1, 1))
    return out
